# baseline (speedup 1.0000x reference)
import functools

import jax
import jax.numpy as jnp
from jax import lax
from jax.experimental import pallas as pl
from jax.experimental.pallas import tpu as pltpu
from jax.experimental.pallas import tpu_sc as plsc

N = 10000
E = 320000
D = 128
NCLS = 40
NC = 2
NS = 16
L = 16

K = 128
DEPTH = 3
EP = 331776
EPT = EP // NS
CH = EPT // K
EPT1 = EP // (NC * NS)
CH1 = EPT1 // K
RB = 80
NBLK = N // RB
NFULL = NBLK // NS
NTAIL = NBLK - NFULL * NS


def _mid_body(p0_ref, p1_ref, b_ref, w1_ref, w2_ref, o_ref):
    sblk = p0_ref[...] + p1_ref[...]
    h = jnp.maximum(jnp.dot(sblk, w1_ref[0], preferred_element_type=jnp.float32)
                    + b_ref[0], 0.0)
    o_ref[...] = jnp.dot(h, w2_ref[0], preferred_element_type=jnp.float32)


_RB_TC = 1000


def _dense_mid(sparts, b_stacked, w1_stacked, w2_stacked):
    nb = N // _RB_TC
    return pl.pallas_call(
        _mid_body,
        grid=(2 * nb,),
        in_specs=[
            pl.BlockSpec((_RB_TC, D), lambda i: (i % (N // _RB_TC), 0)),
            pl.BlockSpec((_RB_TC, D), lambda i: (i % (N // _RB_TC) + N // _RB_TC, 0)),
            pl.BlockSpec((1, 1, D), lambda i: (i // (N // _RB_TC), 0, 0)),
            pl.BlockSpec((1, D, D), lambda i: (i // (N // _RB_TC), 0, 0)),
            pl.BlockSpec((1, D, D), lambda i: (i // (N // _RB_TC), 0, 0)),
        ],
        out_specs=pl.BlockSpec((_RB_TC, D), lambda i: (i, 0)),
        out_shape=jax.ShapeDtypeStruct((2 * N, D), jnp.float32),
    )(sparts, sparts, b_stacked, w1_stacked, w2_stacked)


def _final_body(g_ref, h_ref, b2_ref, bh2_ref, alpha_ref, wc_ref, bc_ref,
                zg_ref, zh_ref, z_ref, lg_ref):
    def l2n(v):
        nrm = jnp.sqrt(jnp.sum(v * v, axis=1, keepdims=True))
        return v / jnp.maximum(nrm, 1e-12)

    zg = l2n(g_ref[...] + b2_ref[...])
    zh = l2n(h_ref[...] + bh2_ref[...])
    a = alpha_ref[0, 0]
    z = l2n(a * zg + (1.0 - a) * zh)
    zg_ref[...] = zg
    zh_ref[...] = zh
    z_ref[...] = z
    lg_ref[...] = jnp.dot(z, wc_ref[...], preferred_element_type=jnp.float32) + bc_ref[...]


def _final(agg2, b2, bh2, alpha, wc, bc):
    nb = N // _RB_TC
    return pl.pallas_call(
        _final_body,
        grid=(nb,),
        in_specs=[
            pl.BlockSpec((_RB_TC, D), lambda i: (i, 0)),
            pl.BlockSpec((_RB_TC, D), lambda i: (i + N // _RB_TC, 0)),
            pl.BlockSpec((1, D), lambda i: (0, 0)),
            pl.BlockSpec((1, D), lambda i: (0, 0)),
            pl.BlockSpec(memory_space=pltpu.SMEM),
            pl.BlockSpec((D, NCLS), lambda i: (0, 0)),
            pl.BlockSpec((1, NCLS), lambda i: (0, 0)),
        ],
        out_specs=[
            pl.BlockSpec((_RB_TC, D), lambda i: (i, 0)),
            pl.BlockSpec((_RB_TC, D), lambda i: (i, 0)),
            pl.BlockSpec((_RB_TC, D), lambda i: (i, 0)),
            pl.BlockSpec((_RB_TC, NCLS), lambda i: (i, 0)),
        ],
        out_shape=[
            jax.ShapeDtypeStruct((N, D), jnp.float32),
            jax.ShapeDtypeStruct((N, D), jnp.float32),
            jax.ShapeDtypeStruct((N, D), jnp.float32),
            jax.ShapeDtypeStruct((N, NCLS), jnp.float32),
        ],
    )(agg2, agg2, b2, bh2, alpha, wc, bc)


@functools.lru_cache(maxsize=2)
def _make_sc_scatter(split_edges):
    mesh = plsc.VectorSubcoreMesh(core_axis_name="c", subcore_axis_name="s")
    n_ch = CH1 if split_edges else CH

    @functools.partial(
        pl.kernel,
        out_type=jax.ShapeDtypeStruct((2 * N, D), jnp.float32),
        mesh=mesh,
        scratch_types=[
            pltpu.VMEM_SHARED((N, D), jnp.float32),
            [pltpu.VMEM((K,), jnp.int32)] * DEPTH,
            [pltpu.VMEM((K,), jnp.int32)] * DEPTH,
            [pltpu.VMEM((K,), jnp.float32)] * DEPTH,
            [pltpu.VMEM((K,), jnp.int32)] * 2,
            [pltpu.VMEM((K, D), jnp.float32)] * DEPTH,
            [pltpu.SemaphoreType.DMA] * DEPTH,
            [pltpu.SemaphoreType.DMA] * DEPTH,
            [pltpu.SemaphoreType.DMA] * DEPTH,
            pltpu.SemaphoreType.DMA,
        ],
    )
    def sc_scatter(src_h, dst_h, ew_h, table, out, acc,
                   srcb, dstb, ewb, db, gb, sg, ss, sm, sw):
        cid = lax.axis_index("c")
        sid = lax.axis_index("s")

        wb = gb[0].at[pl.ds(0, RB)]

        def zrow(r, carry):
            for j in range(D // L):
                gb[0][r, pl.ds(j * L, L)] = jnp.zeros((L,), jnp.float32)
            return carry
        lax.fori_loop(0, RB, zrow, 0)
        for k in range(NFULL):
            pltpu.async_copy(wb, acc.at[pl.ds((sid + k * NS) * RB, RB)], sw)

        @pl.when(sid < NTAIL)
        def _zero_tail():
            pltpu.async_copy(wb, acc.at[pl.ds((sid + NFULL * NS) * RB, RB)], sw)
        for k in range(NFULL):
            pltpu.make_async_copy(wb, acc.at[pl.ds(sid * RB, RB)], sw).wait()

        @pl.when(sid < NTAIL)
        def _zero_tail_wait():
            pltpu.make_async_copy(wb, acc.at[pl.ds(sid * RB, RB)], sw).wait()
        plsc.subcore_barrier()

        if split_edges:
            base = cid * (EP // NC) + sid * EPT1
        else:
            base = sid * EPT
        branch_off = cid * N

        def issue_meta(c, i):
            b = base + c * K
            pltpu.async_copy(src_h.at[pl.ds(b, K)], srcb[i], sm[i])
            pltpu.async_copy(dst_h.at[pl.ds(b, K)], dstb[i], sm[i])
            pltpu.async_copy(ew_h.at[pl.ds(b, K)], ewb[i], sm[i])

        def wait_meta(i):
            pltpu.make_async_copy(src_h.at[pl.ds(base, K)], srcb[i], sm[i]).wait()
            pltpu.make_async_copy(dst_h.at[pl.ds(base, K)], dstb[i], sm[i]).wait()
            pltpu.make_async_copy(ew_h.at[pl.ds(base, K)], ewb[i], sm[i]).wait()

        def adjust_src(i):
            for j in range(K // L):
                srcb[i][pl.ds(j * L, L)] = srcb[i][pl.ds(j * L, L)] + branch_off

        def multiply(i):
            def group(g, icarry):
                wv = ewb[i][pl.ds(g * L, L)]
                for q in range(L):
                    w = wv[q]
                    e = g * L + q
                    for j in range(D // L):
                        gb[i][e, pl.ds(j * L, L)] = gb[i][e, pl.ds(j * L, L)] * w
                return icarry
            lax.fori_loop(0, K // L, group, 0)

        def half(c, p, n, nn):
            @pl.when(c >= 2)
            def _drain():
                pltpu.make_async_copy(gb[n], acc.at[db[n % 2]], ss[n]).wait()

            @pl.when(c + 1 < n_ch)
            def _launch_next():
                wait_meta(n)
                adjust_src(n)
                pltpu.async_copy(table.at[srcb[n]], gb[n], sg[n])

            pltpu.make_async_copy(table.at[srcb[p]], gb[p], sg[p]).wait()
            multiply(p)
            for j in range(K // L):
                db[p % 2][pl.ds(j * L, L)] = dstb[p][pl.ds(j * L, L)]
            pltpu.async_copy(gb[p], acc.at[db[p % 2]], ss[p], add=True)

            @pl.when(c + 3 < n_ch)
            def _prefetch():
                issue_meta(c + 3, p)

        pltpu.sync_copy(src_h.at[pl.ds(base, K)], srcb[0])
        pltpu.sync_copy(dst_h.at[pl.ds(base, K)], dstb[0])
        pltpu.sync_copy(ew_h.at[pl.ds(base, K)], ewb[0])
        adjust_src(0)
        pltpu.async_copy(table.at[srcb[0]], gb[0], sg[0])
        issue_meta(1, 1)
        issue_meta(2, 2)

        def trip(t, carry):
            c = DEPTH * t
            half(c, 0, 1, 2)
            half(c + 1, 1, 2, 0)
            half(c + 2, 2, 0, 1)
            return carry
        lax.fori_loop(0, n_ch // DEPTH, trip, 0)
        pltpu.make_async_copy(gb[(n_ch - 2) % DEPTH],
                              acc.at[db[(n_ch - 2) % 2]],
                              ss[(n_ch - 2) % DEPTH]).wait()
        pltpu.make_async_copy(gb[(n_ch - 1) % DEPTH],
                              acc.at[db[(n_ch - 1) % 2]],
                              ss[(n_ch - 1) % DEPTH]).wait()
        plsc.subcore_barrier()

        for k in range(NFULL):
            r0 = (sid + k * NS) * RB
            pltpu.async_copy(acc.at[pl.ds(r0, RB)],
                             out.at[pl.ds(cid * N + r0, RB)], sw)

        @pl.when(sid < NTAIL)
        def _wb_tail():
            r0 = (sid + NFULL * NS) * RB
            pltpu.async_copy(acc.at[pl.ds(r0, RB)],
                             out.at[pl.ds(cid * N + r0, RB)], sw)
        for k in range(NFULL):
            pltpu.make_async_copy(acc.at[pl.ds(sid * RB, RB)],
                                  out.at[pl.ds(sid * RB, RB)], sw).wait()

        @pl.when(sid < NTAIL)
        def _wb_tail_wait():
            pltpu.make_async_copy(acc.at[pl.ds(sid * RB, RB)],
                                  out.at[pl.ds(sid * RB, RB)], sw).wait()

    return sc_scatter


def kernel(x, edge_index, edge_weight, W1, b1, W2, b2, Wh1, bh1, Wh2, bh2,
           alpha, Wc, bc):
    src = edge_index[0].astype(jnp.int32)
    dst = edge_index[1].astype(jnp.int32)
    pad = EP - E
    srcp = jnp.pad(src, (0, pad))
    dstp = jnp.concatenate([dst, jnp.arange(pad, dtype=jnp.int32) % N])
    ewp = jnp.pad(edge_weight.astype(jnp.float32), (0, pad))

    w1s = jnp.stack([W1, Wh1])
    w2s = jnp.stack([W2, Wh2])
    b1s = jnp.stack([b1, bh1]).reshape(2, 1, D)

    sc_split = _make_sc_scatter(True)
    sc_branch = _make_sc_scatter(False)
    xx = jnp.concatenate([x, x], axis=0)
    sparts = sc_split(srcp, dstp, ewp, xx)
    table2 = _dense_mid(sparts, b1s, w1s, w2s)
    agg2 = sc_branch(srcp, dstp, ewp, table2)
    z_gcn, z_hgcn, z, logits = _final(
        agg2, b2.reshape(1, D), bh2.reshape(1, D),
        alpha.reshape(1, 1), Wc, bc.reshape(1, NCLS))
    return (z_gcn, z_hgcn, z, logits)


# speedup vs baseline: 1.5685x; 1.5685x over previous
import functools

import jax
import jax.numpy as jnp
from jax import lax
from jax.experimental import pallas as pl
from jax.experimental.pallas import tpu as pltpu
from jax.experimental.pallas import tpu_sc as plsc

N = 10000
E = 320000
D = 128
NCLS = 40
NC = 2
NS = 16
L = 16

EPT = 20352
EP = NS * EPT
K = 128
CH = EPT // K
DEPTH = 3
CHT = CH // DEPTH
RB = 80
NBLK = N // RB
NFULL = NBLK // NS
NTAIL = NBLK - NFULL * NS


def _mm_body(x_ref, w_ref, o_ref):
    o_ref[...] = jnp.dot(x_ref[...], w_ref[0], preferred_element_type=jnp.float32)


def _mm_relu_body(x_ref, b_ref, w_ref, o_ref):
    h = jnp.maximum(x_ref[...] + b_ref[0], 0.0)
    o_ref[...] = jnp.dot(h, w_ref[0], preferred_element_type=jnp.float32)


_RB_TC = 1000


def _dense_first(x, w_stacked):
    return pl.pallas_call(
        _mm_body,
        grid=(2 * N // _RB_TC,),
        in_specs=[
            pl.BlockSpec((_RB_TC, D), lambda i: (i % (N // _RB_TC), 0)),
            pl.BlockSpec((1, D, D), lambda i: (i // (N // _RB_TC), 0, 0)),
        ],
        out_specs=pl.BlockSpec((_RB_TC, D), lambda i: (i, 0)),
        out_shape=jax.ShapeDtypeStruct((2 * N, D), jnp.float32),
    )(x, w_stacked)


def _dense_mid(agg, b_stacked, w_stacked):
    return pl.pallas_call(
        _mm_relu_body,
        grid=(2 * N // _RB_TC,),
        in_specs=[
            pl.BlockSpec((_RB_TC, D), lambda i: (i, 0)),
            pl.BlockSpec((1, 1, D), lambda i: (i // (N // _RB_TC), 0, 0)),
            pl.BlockSpec((1, D, D), lambda i: (i // (N // _RB_TC), 0, 0)),
        ],
        out_specs=pl.BlockSpec((_RB_TC, D), lambda i: (i, 0)),
        out_shape=jax.ShapeDtypeStruct((2 * N, D), jnp.float32),
    )(agg, b_stacked, w_stacked)


def _final_body(g_ref, h_ref, b2_ref, bh2_ref, alpha_ref, wc_ref, bc_ref,
                zg_ref, zh_ref, z_ref, lg_ref):
    def l2n(v):
        nrm = jnp.sqrt(jnp.sum(v * v, axis=1, keepdims=True))
        return v / jnp.maximum(nrm, 1e-12)

    zg = l2n(g_ref[...] + b2_ref[...])
    zh = l2n(h_ref[...] + bh2_ref[...])
    a = alpha_ref[0, 0]
    z = l2n(a * zg + (1.0 - a) * zh)
    zg_ref[...] = zg
    zh_ref[...] = zh
    z_ref[...] = z
    lg_ref[...] = jnp.dot(z, wc_ref[...], preferred_element_type=jnp.float32) + bc_ref[...]


def _final(agg2, b2, bh2, alpha, wc, bc):
    nb = N // _RB_TC
    return pl.pallas_call(
        _final_body,
        grid=(nb,),
        in_specs=[
            pl.BlockSpec((_RB_TC, D), lambda i: (i, 0)),
            pl.BlockSpec((_RB_TC, D), lambda i: (i + N // _RB_TC, 0)),
            pl.BlockSpec((1, D), lambda i: (0, 0)),
            pl.BlockSpec((1, D), lambda i: (0, 0)),
            pl.BlockSpec(memory_space=pltpu.SMEM),
            pl.BlockSpec((D, NCLS), lambda i: (0, 0)),
            pl.BlockSpec((1, NCLS), lambda i: (0, 0)),
        ],
        out_specs=[
            pl.BlockSpec((_RB_TC, D), lambda i: (i, 0)),
            pl.BlockSpec((_RB_TC, D), lambda i: (i, 0)),
            pl.BlockSpec((_RB_TC, D), lambda i: (i, 0)),
            pl.BlockSpec((_RB_TC, NCLS), lambda i: (i, 0)),
        ],
        out_shape=[
            jax.ShapeDtypeStruct((N, D), jnp.float32),
            jax.ShapeDtypeStruct((N, D), jnp.float32),
            jax.ShapeDtypeStruct((N, D), jnp.float32),
            jax.ShapeDtypeStruct((N, NCLS), jnp.float32),
        ],
    )(agg2, agg2, b2, bh2, alpha, wc, bc)


@functools.lru_cache(maxsize=1)
def _make_sc_scatter():
    mesh = plsc.VectorSubcoreMesh(core_axis_name="c", subcore_axis_name="s")

    @functools.partial(
        pl.kernel,
        out_type=jax.ShapeDtypeStruct((2 * N, D), jnp.float32),
        mesh=mesh,
        scratch_types=[
            pltpu.VMEM_SHARED((N, D), jnp.float32),
            [pltpu.VMEM((K,), jnp.int32)] * DEPTH,
            [pltpu.VMEM((K,), jnp.int32)] * DEPTH,
            [pltpu.VMEM((K,), jnp.float32)] * DEPTH,
            [pltpu.VMEM((K,), jnp.int32)] * 2,
            [pltpu.VMEM((K, D), jnp.float32)] * DEPTH,
            [pltpu.SemaphoreType.DMA] * DEPTH,
            [pltpu.SemaphoreType.DMA] * DEPTH,
            [pltpu.SemaphoreType.DMA] * DEPTH,
            pltpu.SemaphoreType.DMA,
        ],
    )
    def sc_scatter(src_h, dst_h, ew_h, table, out, acc,
                   srcb, dstb, ewb, db, gb, sg, ss, sm, sw):
        cid = lax.axis_index("c")
        sid = lax.axis_index("s")

        wb = gb[0].at[pl.ds(0, RB)]

        def zrow(r, carry):
            for j in range(D // L):
                gb[0][r, pl.ds(j * L, L)] = jnp.zeros((L,), jnp.float32)
            return carry
        lax.fori_loop(0, RB, zrow, 0)
        for k in range(NFULL):
            pltpu.async_copy(wb, acc.at[pl.ds((sid + k * NS) * RB, RB)], sw)

        @pl.when(sid < NTAIL)
        def _zero_tail():
            pltpu.async_copy(wb, acc.at[pl.ds((sid + NFULL * NS) * RB, RB)], sw)
        for k in range(NFULL):
            pltpu.make_async_copy(wb, acc.at[pl.ds(sid * RB, RB)], sw).wait()

        @pl.when(sid < NTAIL)
        def _zero_tail_wait():
            pltpu.make_async_copy(wb, acc.at[pl.ds(sid * RB, RB)], sw).wait()
        plsc.subcore_barrier()

        base = sid * EPT
        branch_off = cid * N

        def issue_meta(c, i):
            b = base + c * K
            pltpu.async_copy(src_h.at[pl.ds(b, K)], srcb[i], sm[i])
            pltpu.async_copy(dst_h.at[pl.ds(b, K)], dstb[i], sm[i])
            pltpu.async_copy(ew_h.at[pl.ds(b, K)], ewb[i], sm[i])

        def wait_meta(i):
            pltpu.make_async_copy(src_h.at[pl.ds(base, K)], srcb[i], sm[i]).wait()
            pltpu.make_async_copy(dst_h.at[pl.ds(base, K)], dstb[i], sm[i]).wait()
            pltpu.make_async_copy(ew_h.at[pl.ds(base, K)], ewb[i], sm[i]).wait()

        def adjust_src(i):
            for j in range(K // L):
                srcb[i][pl.ds(j * L, L)] = srcb[i][pl.ds(j * L, L)] + branch_off

        def multiply(i):
            def group(g, icarry):
                wv = ewb[i][pl.ds(g * L, L)]
                for q in range(L):
                    w = wv[q]
                    e = g * L + q
                    for j in range(D // L):
                        gb[i][e, pl.ds(j * L, L)] = gb[i][e, pl.ds(j * L, L)] * w
                return icarry
            lax.fori_loop(0, K // L, group, 0)

        def half(c, p, n, nn):
            @pl.when(c >= 2)
            def _drain():
                pltpu.make_async_copy(gb[n], acc.at[db[n % 2]], ss[n]).wait()

            @pl.when(c + 1 < CH)
            def _launch_next():
                wait_meta(n)
                adjust_src(n)
                pltpu.async_copy(table.at[srcb[n]], gb[n], sg[n])

            pltpu.make_async_copy(table.at[srcb[p]], gb[p], sg[p]).wait()
            multiply(p)
            for j in range(K // L):
                db[p % 2][pl.ds(j * L, L)] = dstb[p][pl.ds(j * L, L)]
            pltpu.async_copy(gb[p], acc.at[db[p % 2]], ss[p], add=True)

            @pl.when(c + 3 < CH)
            def _prefetch():
                issue_meta(c + 3, p)

        pltpu.sync_copy(src_h.at[pl.ds(base, K)], srcb[0])
        pltpu.sync_copy(dst_h.at[pl.ds(base, K)], dstb[0])
        pltpu.sync_copy(ew_h.at[pl.ds(base, K)], ewb[0])
        adjust_src(0)
        pltpu.async_copy(table.at[srcb[0]], gb[0], sg[0])
        issue_meta(1, 1)
        issue_meta(2, 2)

        def trip(t, carry):
            c = DEPTH * t
            half(c, 0, 1, 2)
            half(c + 1, 1, 2, 0)
            half(c + 2, 2, 0, 1)
            return carry
        lax.fori_loop(0, CHT, trip, 0)
        pltpu.make_async_copy(gb[(CH - 2) % DEPTH],
                              acc.at[db[(CH - 2) % 2]],
                              ss[(CH - 2) % DEPTH]).wait()
        pltpu.make_async_copy(gb[(CH - 1) % DEPTH],
                              acc.at[db[(CH - 1) % 2]],
                              ss[(CH - 1) % DEPTH]).wait()
        plsc.subcore_barrier()

        for k in range(NFULL):
            r0 = (sid + k * NS) * RB
            pltpu.async_copy(acc.at[pl.ds(r0, RB)],
                             out.at[pl.ds(cid * N + r0, RB)], sw)

        @pl.when(sid < NTAIL)
        def _wb_tail():
            r0 = (sid + NFULL * NS) * RB
            pltpu.async_copy(acc.at[pl.ds(r0, RB)],
                             out.at[pl.ds(cid * N + r0, RB)], sw)
        for k in range(NFULL):
            pltpu.make_async_copy(acc.at[pl.ds(sid * RB, RB)],
                                  out.at[pl.ds(sid * RB, RB)], sw).wait()

        @pl.when(sid < NTAIL)
        def _wb_tail_wait():
            pltpu.make_async_copy(acc.at[pl.ds(sid * RB, RB)],
                                  out.at[pl.ds(sid * RB, RB)], sw).wait()

    return sc_scatter


def kernel(x, edge_index, edge_weight, W1, b1, W2, b2, Wh1, bh1, Wh2, bh2,
           alpha, Wc, bc):
    src = edge_index[0].astype(jnp.int32)
    dst = edge_index[1].astype(jnp.int32)
    pad = EP - E
    srcp = jnp.pad(src, (0, pad))
    dstp = jnp.concatenate([dst, jnp.arange(pad, dtype=jnp.int32) % N])
    ewp = jnp.pad(edge_weight.astype(jnp.float32), (0, pad))

    w1s = jnp.stack([W1, Wh1])
    w2s = jnp.stack([W2, Wh2])
    b1s = jnp.stack([b1, bh1]).reshape(2, 1, D)

    sc_scatter = _make_sc_scatter()
    table1 = _dense_first(x, w1s)
    agg1 = sc_scatter(srcp, dstp, ewp, table1)
    table2 = _dense_mid(agg1, b1s, w2s)
    agg2 = sc_scatter(srcp, dstp, ewp, table2)
    z_gcn, z_hgcn, z, logits = _final(
        agg2, b2.reshape(1, D), bh2.reshape(1, D),
        alpha.reshape(1, 1), Wc, bc.reshape(1, NCLS))
    return (z_gcn, z_hgcn, z, logits)


# speedup vs baseline: 1.5715x; 1.0020x over previous
import functools

import jax
import jax.numpy as jnp
from jax import lax
from jax.experimental import pallas as pl
from jax.experimental.pallas import tpu as pltpu
from jax.experimental.pallas import tpu_sc as plsc

N = 10000
E = 320000
D = 128
NCLS = 40
NC = 2
NS = 16
L = 16

EPT = 20352
EP = NS * EPT
K = 128
CH = EPT // K
DEPTH = 3
CHT = CH // DEPTH
RB = 80
NBLK = N // RB
NFULL = NBLK // NS
NTAIL = NBLK - NFULL * NS


def _mm_body(x_ref, w_ref, o_ref):
    o_ref[...] = jnp.dot(x_ref[...], w_ref[0], preferred_element_type=jnp.float32)


def _mm_relu_body(x_ref, b_ref, w_ref, o_ref):
    h = jnp.maximum(x_ref[...] + b_ref[0], 0.0)
    o_ref[...] = jnp.dot(h, w_ref[0], preferred_element_type=jnp.float32)


_RB_TC = 1000


def _dense_first(x, w_stacked):
    return pl.pallas_call(
        _mm_body,
        grid=(2 * N // _RB_TC,),
        in_specs=[
            pl.BlockSpec((_RB_TC, D), lambda i: (i % (N // _RB_TC), 0)),
            pl.BlockSpec((1, D, D), lambda i: (i // (N // _RB_TC), 0, 0)),
        ],
        out_specs=pl.BlockSpec((_RB_TC, D), lambda i: (i, 0)),
        out_shape=jax.ShapeDtypeStruct((2 * N, D), jnp.float32),
    )(x, w_stacked)


def _dense_mid(agg, b_stacked, w_stacked):
    return pl.pallas_call(
        _mm_relu_body,
        grid=(2 * N // _RB_TC,),
        in_specs=[
            pl.BlockSpec((_RB_TC, D), lambda i: (i, 0)),
            pl.BlockSpec((1, 1, D), lambda i: (i // (N // _RB_TC), 0, 0)),
            pl.BlockSpec((1, D, D), lambda i: (i // (N // _RB_TC), 0, 0)),
        ],
        out_specs=pl.BlockSpec((_RB_TC, D), lambda i: (i, 0)),
        out_shape=jax.ShapeDtypeStruct((2 * N, D), jnp.float32),
    )(agg, b_stacked, w_stacked)


def _final_body(g_ref, h_ref, b2_ref, bh2_ref, alpha_ref, wc_ref, bc_ref,
                zg_ref, zh_ref, z_ref, lg_ref):
    def l2n(v):
        nrm = jnp.sqrt(jnp.sum(v * v, axis=1, keepdims=True))
        return v / jnp.maximum(nrm, 1e-12)

    zg = l2n(g_ref[...] + b2_ref[...])
    zh = l2n(h_ref[...] + bh2_ref[...])
    a = alpha_ref[0, 0]
    z = l2n(a * zg + (1.0 - a) * zh)
    zg_ref[...] = zg
    zh_ref[...] = zh
    z_ref[...] = z
    lg_ref[...] = jnp.dot(z, wc_ref[...], preferred_element_type=jnp.float32) + bc_ref[...]


def _final(agg2, b2, bh2, alpha, wc, bc):
    nb = N // _RB_TC
    return pl.pallas_call(
        _final_body,
        grid=(nb,),
        in_specs=[
            pl.BlockSpec((_RB_TC, D), lambda i: (i, 0)),
            pl.BlockSpec((_RB_TC, D), lambda i: (i + N // _RB_TC, 0)),
            pl.BlockSpec((1, D), lambda i: (0, 0)),
            pl.BlockSpec((1, D), lambda i: (0, 0)),
            pl.BlockSpec(memory_space=pltpu.SMEM),
            pl.BlockSpec((D, NCLS), lambda i: (0, 0)),
            pl.BlockSpec((1, NCLS), lambda i: (0, 0)),
        ],
        out_specs=[
            pl.BlockSpec((_RB_TC, D), lambda i: (i, 0)),
            pl.BlockSpec((_RB_TC, D), lambda i: (i, 0)),
            pl.BlockSpec((_RB_TC, D), lambda i: (i, 0)),
            pl.BlockSpec((_RB_TC, NCLS), lambda i: (i, 0)),
        ],
        out_shape=[
            jax.ShapeDtypeStruct((N, D), jnp.float32),
            jax.ShapeDtypeStruct((N, D), jnp.float32),
            jax.ShapeDtypeStruct((N, D), jnp.float32),
            jax.ShapeDtypeStruct((N, NCLS), jnp.float32),
        ],
    )(agg2, agg2, b2, bh2, alpha, wc, bc)


@functools.lru_cache(maxsize=1)
def _make_sc_scatter():
    mesh = plsc.VectorSubcoreMesh(core_axis_name="c", subcore_axis_name="s")

    @functools.partial(
        pl.kernel,
        out_type=jax.ShapeDtypeStruct((2 * N, D), jnp.float32),
        mesh=mesh,
        scratch_types=[
            pltpu.VMEM_SHARED((N, D), jnp.float32),
            [pltpu.VMEM((2, K), jnp.int32)] * DEPTH,
            [pltpu.VMEM((K,), jnp.float32)] * DEPTH,
            [pltpu.VMEM((K,), jnp.int32)] * 2,
            [pltpu.VMEM((K, D), jnp.float32)] * DEPTH,
            [pltpu.SemaphoreType.DMA] * DEPTH,
            [pltpu.SemaphoreType.DMA] * DEPTH,
            [pltpu.SemaphoreType.DMA] * DEPTH,
            pltpu.SemaphoreType.DMA,
        ],
    )
    def sc_scatter(sd_h, ew_h, table, out, acc,
                   sdb, ewb, db, gb, sg, ss, sm, sw):
        cid = lax.axis_index("c")
        sid = lax.axis_index("s")

        wb = gb[0].at[pl.ds(0, RB)]

        def zrow(r, carry):
            for j in range(D // L):
                gb[0][r, pl.ds(j * L, L)] = jnp.zeros((L,), jnp.float32)
            return carry
        lax.fori_loop(0, RB, zrow, 0)
        for k in range(NFULL):
            pltpu.async_copy(wb, acc.at[pl.ds((sid + k * NS) * RB, RB)], sw)

        @pl.when(sid < NTAIL)
        def _zero_tail():
            pltpu.async_copy(wb, acc.at[pl.ds((sid + NFULL * NS) * RB, RB)], sw)
        for k in range(NFULL):
            pltpu.make_async_copy(wb, acc.at[pl.ds(sid * RB, RB)], sw).wait()

        @pl.when(sid < NTAIL)
        def _zero_tail_wait():
            pltpu.make_async_copy(wb, acc.at[pl.ds(sid * RB, RB)], sw).wait()
        plsc.subcore_barrier()

        base = sid * EPT
        branch_off = cid * N

        def issue_meta(c, i):
            b = base + c * K
            pltpu.async_copy(sd_h.at[:, pl.ds(b, K)], sdb[i], sm[i])
            pltpu.async_copy(ew_h.at[pl.ds(b, K)], ewb[i], sm[i])

        def wait_meta(i):
            pltpu.make_async_copy(sd_h.at[:, pl.ds(base, K)], sdb[i], sm[i]).wait()
            pltpu.make_async_copy(ew_h.at[pl.ds(base, K)], ewb[i], sm[i]).wait()

        def adjust_src(i):
            for j in range(K // L):
                sdb[i][0, pl.ds(j * L, L)] = sdb[i][0, pl.ds(j * L, L)] + branch_off

        def multiply(i):
            def group(g, icarry):
                wv = ewb[i][pl.ds(g * L, L)]
                for q in range(L):
                    w = wv[q]
                    e = g * L + q
                    for j in range(D // L):
                        gb[i][e, pl.ds(j * L, L)] = gb[i][e, pl.ds(j * L, L)] * w
                return icarry
            lax.fori_loop(0, K // L, group, 0)

        def half(c, p, n, nn):
            @pl.when(c >= 2)
            def _drain():
                pltpu.make_async_copy(gb[n], acc.at[db[n % 2]], ss[n]).wait()

            @pl.when(c + 1 < CH)
            def _launch_next():
                wait_meta(n)
                adjust_src(n)
                pltpu.async_copy(table.at[sdb[n].at[0]], gb[n], sg[n])

            pltpu.make_async_copy(table.at[sdb[p].at[0]], gb[p], sg[p]).wait()
            multiply(p)
            for j in range(K // L):
                db[p % 2][pl.ds(j * L, L)] = sdb[p][1, pl.ds(j * L, L)]
            pltpu.async_copy(gb[p], acc.at[db[p % 2]], ss[p], add=True)

            @pl.when(c + 3 < CH)
            def _prefetch():
                issue_meta(c + 3, p)

        pltpu.sync_copy(sd_h.at[:, pl.ds(base, K)], sdb[0])
        pltpu.sync_copy(ew_h.at[pl.ds(base, K)], ewb[0])
        adjust_src(0)
        pltpu.async_copy(table.at[sdb[0].at[0]], gb[0], sg[0])
        issue_meta(1, 1)
        issue_meta(2, 2)

        def trip(t, carry):
            c = DEPTH * t
            half(c, 0, 1, 2)
            half(c + 1, 1, 2, 0)
            half(c + 2, 2, 0, 1)
            return carry
        lax.fori_loop(0, CHT, trip, 0)
        pltpu.make_async_copy(gb[(CH - 2) % DEPTH],
                              acc.at[db[(CH - 2) % 2]],
                              ss[(CH - 2) % DEPTH]).wait()
        pltpu.make_async_copy(gb[(CH - 1) % DEPTH],
                              acc.at[db[(CH - 1) % 2]],
                              ss[(CH - 1) % DEPTH]).wait()
        plsc.subcore_barrier()

        for k in range(NFULL):
            r0 = (sid + k * NS) * RB
            pltpu.async_copy(acc.at[pl.ds(r0, RB)],
                             out.at[pl.ds(cid * N + r0, RB)], sw)

        @pl.when(sid < NTAIL)
        def _wb_tail():
            r0 = (sid + NFULL * NS) * RB
            pltpu.async_copy(acc.at[pl.ds(r0, RB)],
                             out.at[pl.ds(cid * N + r0, RB)], sw)
        for k in range(NFULL):
            pltpu.make_async_copy(acc.at[pl.ds(sid * RB, RB)],
                                  out.at[pl.ds(sid * RB, RB)], sw).wait()

        @pl.when(sid < NTAIL)
        def _wb_tail_wait():
            pltpu.make_async_copy(acc.at[pl.ds(sid * RB, RB)],
                                  out.at[pl.ds(sid * RB, RB)], sw).wait()

    return sc_scatter


def kernel(x, edge_index, edge_weight, W1, b1, W2, b2, Wh1, bh1, Wh2, bh2,
           alpha, Wc, bc):
    src = edge_index[0].astype(jnp.int32)
    dst = edge_index[1].astype(jnp.int32)
    pad = EP - E
    srcp = jnp.pad(src, (0, pad))
    dstp = jnp.concatenate([dst, jnp.arange(pad, dtype=jnp.int32) % N])
    ewp = jnp.pad(edge_weight.astype(jnp.float32), (0, pad))

    w1s = jnp.stack([W1, Wh1])
    w2s = jnp.stack([W2, Wh2])
    b1s = jnp.stack([b1, bh1]).reshape(2, 1, D)

    sd = jnp.stack([srcp, dstp])
    sc_scatter = _make_sc_scatter()
    table1 = _dense_first(x, w1s)
    agg1 = sc_scatter(sd, ewp, table1)
    table2 = _dense_mid(agg1, b1s, w2s)
    agg2 = sc_scatter(sd, ewp, table2)
    z_gcn, z_hgcn, z, logits = _final(
        agg2, b2.reshape(1, D), bh2.reshape(1, D),
        alpha.reshape(1, 1), Wc, bc.reshape(1, NCLS))
    return (z_gcn, z_hgcn, z, logits)


# speedup vs baseline: 1.5748x; 1.0021x over previous
import functools

import jax
import jax.numpy as jnp
from jax import lax
from jax.experimental import pallas as pl
from jax.experimental.pallas import tpu as pltpu
from jax.experimental.pallas import tpu_sc as plsc

N = 10000
E = 320000
D = 128
NCLS = 40
NC = 2
NS = 16
L = 16

EPT = 20352
EP = NS * EPT
K = 128
CH = EPT // K
DEPTH = 3
CHT = CH // DEPTH
RB = 80
NBLK = N // RB
NFULL = NBLK // NS
NTAIL = NBLK - NFULL * NS


def _mm_body(x_ref, w_ref, o_ref):
    o_ref[...] = jnp.dot(x_ref[...], w_ref[0], preferred_element_type=jnp.float32)


def _mm_relu_body(x_ref, b_ref, w_ref, o_ref):
    h = jnp.maximum(x_ref[...] + b_ref[0], 0.0)
    o_ref[...] = jnp.dot(h, w_ref[0], preferred_element_type=jnp.float32)


_RB_TC = 1000


def _dense_first(x, w_stacked):
    return pl.pallas_call(
        _mm_body,
        grid=(2 * N // _RB_TC,),
        in_specs=[
            pl.BlockSpec((_RB_TC, D), lambda i: (i % (N // _RB_TC), 0)),
            pl.BlockSpec((1, D, D), lambda i: (i // (N // _RB_TC), 0, 0)),
        ],
        out_specs=pl.BlockSpec((_RB_TC, D), lambda i: (i, 0)),
        out_shape=jax.ShapeDtypeStruct((2 * N, D), jnp.float32),
    )(x, w_stacked)


def _dense_mid(agg, b_stacked, w_stacked):
    return pl.pallas_call(
        _mm_relu_body,
        grid=(2 * N // _RB_TC,),
        in_specs=[
            pl.BlockSpec((_RB_TC, D), lambda i: (i, 0)),
            pl.BlockSpec((1, 1, D), lambda i: (i // (N // _RB_TC), 0, 0)),
            pl.BlockSpec((1, D, D), lambda i: (i // (N // _RB_TC), 0, 0)),
        ],
        out_specs=pl.BlockSpec((_RB_TC, D), lambda i: (i, 0)),
        out_shape=jax.ShapeDtypeStruct((2 * N, D), jnp.float32),
    )(agg, b_stacked, w_stacked)


def _final_body(g_ref, h_ref, b2_ref, bh2_ref, alpha_ref, wc_ref, bc_ref,
                zg_ref, zh_ref, z_ref, lg_ref):
    def l2n(v):
        nrm = jnp.sqrt(jnp.sum(v * v, axis=1, keepdims=True))
        return v / jnp.maximum(nrm, 1e-12)

    zg = l2n(g_ref[...] + b2_ref[...])
    zh = l2n(h_ref[...] + bh2_ref[...])
    a = alpha_ref[0, 0]
    z = l2n(a * zg + (1.0 - a) * zh)
    zg_ref[...] = zg
    zh_ref[...] = zh
    z_ref[...] = z
    lg_ref[...] = jnp.dot(z, wc_ref[...], preferred_element_type=jnp.float32) + bc_ref[...]


def _final(agg2, b2, bh2, alpha, wc, bc):
    nb = N // _RB_TC
    return pl.pallas_call(
        _final_body,
        grid=(nb,),
        in_specs=[
            pl.BlockSpec((_RB_TC, D), lambda i: (i, 0)),
            pl.BlockSpec((_RB_TC, D), lambda i: (i + N // _RB_TC, 0)),
            pl.BlockSpec((1, D), lambda i: (0, 0)),
            pl.BlockSpec((1, D), lambda i: (0, 0)),
            pl.BlockSpec(memory_space=pltpu.SMEM),
            pl.BlockSpec((D, NCLS), lambda i: (0, 0)),
            pl.BlockSpec((1, NCLS), lambda i: (0, 0)),
        ],
        out_specs=[
            pl.BlockSpec((_RB_TC, D), lambda i: (i, 0)),
            pl.BlockSpec((_RB_TC, D), lambda i: (i, 0)),
            pl.BlockSpec((_RB_TC, D), lambda i: (i, 0)),
            pl.BlockSpec((_RB_TC, NCLS), lambda i: (i, 0)),
        ],
        out_shape=[
            jax.ShapeDtypeStruct((N, D), jnp.float32),
            jax.ShapeDtypeStruct((N, D), jnp.float32),
            jax.ShapeDtypeStruct((N, D), jnp.float32),
            jax.ShapeDtypeStruct((N, NCLS), jnp.float32),
        ],
    )(agg2, agg2, b2, bh2, alpha, wc, bc)


@functools.lru_cache(maxsize=1)
def _make_sc_scatter():
    mesh = plsc.VectorSubcoreMesh(core_axis_name="c", subcore_axis_name="s")

    @functools.partial(
        pl.kernel,
        out_type=jax.ShapeDtypeStruct((2 * N, D), jnp.float32),
        mesh=mesh,
        scratch_types=[
            pltpu.VMEM_SHARED((N, D), jnp.float32),
            [pltpu.VMEM((K,), jnp.int32)] * DEPTH,
            [pltpu.VMEM((K,), jnp.int32)] * DEPTH,
            [pltpu.VMEM((K,), jnp.float32)] * DEPTH,
            [pltpu.VMEM((K,), jnp.int32)] * 2,
            [pltpu.VMEM((K, D), jnp.float32)] * DEPTH,
            [pltpu.SemaphoreType.DMA] * DEPTH,
            [pltpu.SemaphoreType.DMA] * DEPTH,
            [pltpu.SemaphoreType.DMA] * DEPTH,
            pltpu.SemaphoreType.DMA,
        ],
    )
    def sc_scatter(src_h, dst_h, ew_h, table, out, acc,
                   srcb, dstb, ewb, db, gb, sg, ss, sm, sw):
        cid = lax.axis_index("c")
        sid = lax.axis_index("s")

        wb = gb[0].at[pl.ds(0, RB)]

        def zrow(r, carry):
            for j in range(D // L):
                gb[0][r, pl.ds(j * L, L)] = jnp.zeros((L,), jnp.float32)
            return carry
        lax.fori_loop(0, RB, zrow, 0)
        for k in range(NFULL):
            pltpu.async_copy(wb, acc.at[pl.ds((sid + k * NS) * RB, RB)], sw)

        @pl.when(sid < NTAIL)
        def _zero_tail():
            pltpu.async_copy(wb, acc.at[pl.ds((sid + NFULL * NS) * RB, RB)], sw)
        for k in range(NFULL):
            pltpu.make_async_copy(wb, acc.at[pl.ds(sid * RB, RB)], sw).wait()

        @pl.when(sid < NTAIL)
        def _zero_tail_wait():
            pltpu.make_async_copy(wb, acc.at[pl.ds(sid * RB, RB)], sw).wait()
        plsc.subcore_barrier()

        base = sid * EPT
        branch_off = cid * N

        def issue_meta(c, i):
            b = base + c * K
            pltpu.async_copy(src_h.at[pl.ds(b, K)], srcb[i], sm[i])
            pltpu.async_copy(dst_h.at[pl.ds(b, K)], dstb[i], sm[i])
            pltpu.async_copy(ew_h.at[pl.ds(b, K)], ewb[i], sm[i])

        def wait_meta(i):
            pltpu.make_async_copy(src_h.at[pl.ds(base, K)], srcb[i], sm[i]).wait()
            pltpu.make_async_copy(dst_h.at[pl.ds(base, K)], dstb[i], sm[i]).wait()
            pltpu.make_async_copy(ew_h.at[pl.ds(base, K)], ewb[i], sm[i]).wait()

        def adjust_src(i):
            for j in range(K // L):
                srcb[i][pl.ds(j * L, L)] = srcb[i][pl.ds(j * L, L)] + branch_off

        def multiply(i):
            def group(g, icarry):
                wv = ewb[i][pl.ds(g * L, L)]
                for q in range(L):
                    w = wv[q]
                    e = g * L + q
                    for j in range(D // L):
                        gb[i][e, pl.ds(j * L, L)] = gb[i][e, pl.ds(j * L, L)] * w
                return icarry
            lax.fori_loop(0, K // L, group, 0)

        def half(c, p, n, nn):
            @pl.when(c >= 2)
            def _drain():
                pltpu.make_async_copy(gb[n], acc.at[db[n % 2]], ss[n]).wait()

            @pl.when(c + 1 < CH)
            def _launch_next():
                wait_meta(n)
                adjust_src(n)
                pltpu.async_copy(table.at[srcb[n]], gb[n], sg[n])

            pltpu.make_async_copy(table.at[srcb[p]], gb[p], sg[p]).wait()
            multiply(p)
            for j in range(K // L):
                db[p % 2][pl.ds(j * L, L)] = dstb[p][pl.ds(j * L, L)]
            pltpu.async_copy(gb[p], acc.at[db[p % 2]], ss[p], add=True)

            @pl.when(c + 3 < CH)
            def _prefetch():
                issue_meta(c + 3, p)

        pltpu.sync_copy(src_h.at[pl.ds(base, K)], srcb[0])
        pltpu.sync_copy(dst_h.at[pl.ds(base, K)], dstb[0])
        pltpu.sync_copy(ew_h.at[pl.ds(base, K)], ewb[0])
        adjust_src(0)
        pltpu.async_copy(table.at[srcb[0]], gb[0], sg[0])
        issue_meta(1, 1)
        issue_meta(2, 2)

        def trip(t, carry):
            c = DEPTH * t
            half(c, 0, 1, 2)
            half(c + 1, 1, 2, 0)
            half(c + 2, 2, 0, 1)
            return carry
        lax.fori_loop(0, CHT, trip, 0)
        pltpu.make_async_copy(gb[(CH - 2) % DEPTH],
                              acc.at[db[(CH - 2) % 2]],
                              ss[(CH - 2) % DEPTH]).wait()
        pltpu.make_async_copy(gb[(CH - 1) % DEPTH],
                              acc.at[db[(CH - 1) % 2]],
                              ss[(CH - 1) % DEPTH]).wait()
        plsc.subcore_barrier()

        for k in range(NFULL):
            r0 = (sid + k * NS) * RB
            pltpu.async_copy(acc.at[pl.ds(r0, RB)],
                             out.at[pl.ds(cid * N + r0, RB)], sw)

        @pl.when(sid < NTAIL)
        def _wb_tail():
            r0 = (sid + NFULL * NS) * RB
            pltpu.async_copy(acc.at[pl.ds(r0, RB)],
                             out.at[pl.ds(cid * N + r0, RB)], sw)
        for k in range(NFULL):
            pltpu.make_async_copy(acc.at[pl.ds(sid * RB, RB)],
                                  out.at[pl.ds(sid * RB, RB)], sw).wait()

        @pl.when(sid < NTAIL)
        def _wb_tail_wait():
            pltpu.make_async_copy(acc.at[pl.ds(sid * RB, RB)],
                                  out.at[pl.ds(sid * RB, RB)], sw).wait()

    return sc_scatter


def kernel(x, edge_index, edge_weight, W1, b1, W2, b2, Wh1, bh1, Wh2, bh2,
           alpha, Wc, bc):
    src = edge_index[0].astype(jnp.int32)
    dst = edge_index[1].astype(jnp.int32)
    pad = EP - E
    srcp = jnp.pad(src, (0, pad))
    dstp = jnp.concatenate([dst, jnp.arange(pad, dtype=jnp.int32) % N])
    ewp = jnp.pad(edge_weight.astype(jnp.float32), (0, pad))

    w1s = jnp.stack([W1, Wh1])
    w2s = jnp.stack([W2, Wh2])
    b1s = jnp.stack([b1, bh1]).reshape(2, 1, D)

    sc_scatter = _make_sc_scatter()
    table1 = _dense_first(x, w1s)
    agg1 = sc_scatter(srcp, dstp, ewp, table1)
    table2 = _dense_mid(agg1, b1s, w2s)
    agg2 = sc_scatter(srcp, dstp, ewp, table2)
    z_gcn, z_hgcn, z, logits = _final(
        agg2, b2.reshape(1, D), bh2.reshape(1, D),
        alpha.reshape(1, 1), Wc, bc.reshape(1, NCLS))
    return (z_gcn, z_hgcn, z, logits)
